# baseline (speedup 1.0000x reference)
import functools
import math

import jax
import jax.numpy as jnp
from jax import lax
from jax.experimental import pallas as pl
from jax.experimental.pallas import tpu as pltpu
from jax.experimental.pallas import tpu_sc as plsc

_CHUNK = 128
_NBUF = 4
_LANES = 16


@functools.lru_cache(maxsize=None)
def _build(seq: int, batch: int, vocab: int, dim: int):
    assert batch % _CHUNK == 0 and dim % _LANES == 0
    n_chunks = (seq * batch) // _CHUNK
    nw = 32
    assert n_chunks % nw == 0
    cpw = n_chunks // nw
    assert cpw % _NBUF == 0
    chunks_per_s = batch // _CHUNK
    scale = math.sqrt(dim)
    nk = dim // _LANES

    mesh = plsc.VectorSubcoreMesh(core_axis_name="c", subcore_axis_name="s")

    @functools.partial(
        pl.kernel,
        out_type=jax.ShapeDtypeStruct((n_chunks, _CHUNK, dim), jnp.float32),
        mesh=mesh,
        compiler_params=pltpu.CompilerParams(use_tc_tiling_on_sc=False),
        scratch_types=[
            pltpu.VMEM((cpw, _CHUNK), jnp.int32),
            pltpu.VMEM((seq, dim), jnp.float32),
            pltpu.VMEM((_NBUF, _CHUNK, dim), jnp.float32),
            pltpu.VMEM((_NBUF, _CHUNK, dim), jnp.float32),
            pltpu.SemaphoreType.DMA,
            pltpu.SemaphoreType.DMA,
        ],
    )
    def emb_kernel(x_hbm, table_hbm, pe_hbm, out_hbm,
                   idx_v, pe_v, gbuf, obuf, gsem, osem):
        wid = lax.axis_index("s") * 2 + lax.axis_index("c")
        base = wid * cpw

        pltpu.sync_copy(x_hbm.at[pl.ds(base, cpw)], idx_v)
        pltpu.sync_copy(pe_hbm, pe_v)

        for b in range(_NBUF):
            pltpu.async_copy(table_hbm.at[idx_v.at[b]], gbuf.at[b], gsem)

        def group(g, carry):
            for b in range(_NBUF):
                j = g * _NBUF + b
                c = base + j
                pltpu.make_async_copy(
                    table_hbm.at[idx_v.at[j]], gbuf.at[b], gsem).wait()
                @pl.when(j >= _NBUF)
                def _():
                    pltpu.make_async_copy(
                        obuf.at[b], out_hbm.at[c - _NBUF], osem).wait()

                s = (base + j) // chunks_per_s
                pe_regs = [pe_v[s, pl.ds(k * _LANES, _LANES)]
                           for k in range(nk)]
                g_ref = gbuf.at[b]
                o_ref = obuf.at[b]

                def row(i, carry2):
                    for k in range(nk):
                        sl = pl.ds(k * _LANES, _LANES)
                        o_ref[i, sl] = g_ref[i, sl] * scale + pe_regs[k]
                    return carry2
                lax.fori_loop(0, _CHUNK, row, 0, unroll=2)

                pltpu.async_copy(obuf.at[b], out_hbm.at[c], osem)

                @pl.when(j + _NBUF < cpw)
                def _():
                    pltpu.async_copy(
                        table_hbm.at[idx_v.at[j + _NBUF]], gbuf.at[b], gsem)
            return carry

        lax.fori_loop(0, cpw // _NBUF, group, 0)

        for b in range(_NBUF):
            j = cpw - _NBUF + b
            pltpu.make_async_copy(
                obuf.at[b], out_hbm.at[base + j], osem).wait()

    return emb_kernel


def kernel(x, table, pe, pos):
    seq, batch = x.shape
    vocab, dim = table.shape
    x2 = x.reshape((seq * batch) // _CHUNK, _CHUNK).astype(jnp.int32)
    pe_rows = lax.dynamic_slice_in_dim(pe, pos, seq, axis=0)
    pe_rows = pe_rows.reshape(seq, dim)
    out = _build(seq, batch, vocab, dim)(x2, table, pe_rows)
    return out.reshape(seq, batch, dim)
